# baseline (speedup 1.0000x reference)
import functools

import jax
import jax.numpy as jnp
from jax.experimental import pallas as pl
from jax.experimental.pallas import tpu as pltpu

_BN_EPS = 1e-5
_VMEM_LIMIT = 56 << 20


def _stats_kernel(x_ref, w_ref, sum_ref, ssq_ref, *, nb):
    @pl.when(pl.program_id(1) == 0)
    def _():
        sum_ref[...] = jnp.zeros_like(sum_ref)
        ssq_ref[...] = jnp.zeros_like(ssq_ref)

    w = w_ref[...]
    s = None
    q = None
    for b in range(nb):
        x = x_ref[b].astype(jnp.bfloat16)
        y = jnp.dot(w, x, preferred_element_type=jnp.float32)
        sb = jnp.sum(y, axis=1, keepdims=True)
        qb = jnp.sum(y * y, axis=1, keepdims=True)
        s = sb if s is None else s + sb
        q = qb if q is None else q + qb
    sum_ref[0] += s
    ssq_ref[0] += q


def _apply_kernel(x_ref, w_ref, psum_ref, pssq_ref, g_ref, b_ref, o_ref,
                  *, nb, r):
    mean = jnp.sum(psum_ref[...], axis=0) / r
    var = jnp.maximum(jnp.sum(pssq_ref[...], axis=0) / r - mean * mean, 0.0)
    scale = g_ref[...] * jax.lax.rsqrt(var + _BN_EPS)
    shift = b_ref[...] - mean * scale
    w = (w_ref[...] * scale).astype(jnp.bfloat16)
    for b in range(nb):
        x32 = x_ref[b]
        y = jnp.dot(w, x32.astype(jnp.bfloat16),
                    preferred_element_type=jnp.float32)
        o_ref[b] = jnp.maximum(y + shift, 0.0) + x32


def kernel(x, conv_w, conv_b, bn_gamma, bn_beta):
    del conv_b
    N, C_in, L = x.shape
    C_out = conv_w.shape[0]
    w32 = conv_w[:, :, 0].astype(jnp.float32)

    p = 2 if N % 2 == 0 else 1
    nb0 = next(b for b in (2, 1) if N % (p * b) == 0)
    nb1 = next(b for b in (2, 1) if N % b == 0)
    steps0 = N // (p * nb0)

    w16_spec = pl.BlockSpec((C_out, C_in), lambda *_: (0, 0))
    x_spec0 = pl.BlockSpec((nb0, C_in, L),
                           lambda i, j: (i * steps0 + j, 0, 0))
    stat_spec = pl.BlockSpec((1, C_out, 1), lambda i, j: (i, 0, 0))
    psum, pssq = pl.pallas_call(
        functools.partial(_stats_kernel, nb=nb0),
        out_shape=(jax.ShapeDtypeStruct((p, C_out, 1), jnp.float32),
                   jax.ShapeDtypeStruct((p, C_out, 1), jnp.float32)),
        grid=(p, steps0),
        in_specs=[x_spec0, w16_spec],
        out_specs=(stat_spec, stat_spec),
        compiler_params=pltpu.CompilerParams(
            dimension_semantics=("parallel", "arbitrary"),
            vmem_limit_bytes=_VMEM_LIMIT),
    )(x, w32.astype(jnp.bfloat16))

    x_spec1 = pl.BlockSpec((nb1, C_in, L), lambda n: (n, 0, 0))
    w32_spec = pl.BlockSpec((C_out, C_in), lambda n: (0, 0))
    part_spec = pl.BlockSpec((p, C_out, 1), lambda n: (0, 0, 0))
    vec_spec = pl.BlockSpec((C_out, 1), lambda n: (0, 0))
    out = pl.pallas_call(
        functools.partial(_apply_kernel, nb=nb1, r=float(N * L)),
        out_shape=jax.ShapeDtypeStruct((N, C_out, L), x.dtype),
        grid=(N // nb1,),
        in_specs=[x_spec1, w32_spec, part_spec, part_spec, vec_spec,
                  vec_spec],
        out_specs=pl.BlockSpec((nb1, C_out, L), lambda n: (n, 0, 0)),
        compiler_params=pltpu.CompilerParams(
            dimension_semantics=("parallel",),
            vmem_limit_bytes=_VMEM_LIMIT),
    )(x, w32, psum, pssq, bn_gamma.reshape(C_out, 1),
      bn_beta.reshape(C_out, 1))
    return out


# speedup vs baseline: 1.2640x; 1.2640x over previous
import functools

import jax
import jax.numpy as jnp
from jax.experimental import pallas as pl
from jax.experimental.pallas import tpu as pltpu

_BN_EPS = 1e-5
_VMEM_LIMIT = 58 << 20


def _fused_kernel(x_ref, w_ref, g_ref, b_ref, o_ref,
                  xs_ref, gacc_ref, wsc_ref, shift_ref, *, nb, c, r):
    ph = pl.program_id(0)
    s = pl.program_id(1)

    @pl.when(ph == 0)
    def _phase0():
        xs = []
        for b in range(nb):
            x16 = x_ref[b].astype(jnp.bfloat16)
            xs_ref[pl.ds(s * nb + b, 1)] = x16[None]
            xs.append(x16)
        xx = jnp.concatenate(xs, axis=1) if nb > 1 else xs[0]
        ones = jnp.ones((8, xx.shape[1]), jnp.bfloat16)
        xaug = jnp.concatenate([xx, ones], axis=0)
        g = jax.lax.dot_general(xaug, xx, (((1,), (1,)), ((), ())),
                                preferred_element_type=jnp.float32)

        @pl.when(s == 0)
        def _():
            gacc_ref[...] = g

        @pl.when(s != 0)
        def _():
            gacc_ref[...] += g

    @pl.when((ph == 1) & (s == 0))
    def _fold():
        w32 = w_ref[...]
        w16 = w32.astype(jnp.bfloat16)
        gram = gacc_ref[...]
        wga = jax.lax.dot_general(
            w16, gram.astype(jnp.bfloat16),
            (((1,), (1,)), ((), ())), preferred_element_type=jnp.float32)
        mean = wga[:, c:c + 1] / r
        ey2 = jnp.sum(wga[:, :c] * w32, axis=1, keepdims=True) / r
        var = jnp.maximum(ey2 - mean * mean, 0.0)
        scale = g_ref[...] * jax.lax.rsqrt(var + _BN_EPS)
        shift_ref[...] = b_ref[...] - mean * scale
        wsc_ref[...] = (w32 * scale).astype(jnp.bfloat16)

    @pl.when(ph == 1)
    def _phase1():
        wsc = wsc_ref[...]
        shift = shift_ref[...]
        for b in range(nb):
            x16 = xs_ref[s * nb + b]
            y = jnp.dot(wsc, x16, preferred_element_type=jnp.float32)
            o_ref[b] = jnp.maximum(y + shift, 0.0) + x16.astype(jnp.float32)


def kernel(x, conv_w, conv_b, bn_gamma, bn_beta):
    del conv_b
    N, C_in, L = x.shape
    C_out = conv_w.shape[0]
    w32 = conv_w[:, :, 0].astype(jnp.float32)

    nb = 2 if N % 2 == 0 else 1
    steps = N // nb

    x_spec = pl.BlockSpec(
        (nb, C_in, L), lambda ph, s: (jnp.where(ph == 0, s, steps - 1), 0, 0))
    o_spec = pl.BlockSpec(
        (nb, C_out, L), lambda ph, s: (jnp.where(ph == 0, 0, s), 0, 0))
    w_spec = pl.BlockSpec((C_out, C_in), lambda ph, s: (0, 0))
    vec_spec = pl.BlockSpec((C_out, 1), lambda ph, s: (0, 0))

    out = pl.pallas_call(
        functools.partial(_fused_kernel, nb=nb, c=C_in, r=float(N * L)),
        out_shape=jax.ShapeDtypeStruct((N, C_out, L), x.dtype),
        grid=(2, steps),
        in_specs=[x_spec, w_spec, vec_spec, vec_spec],
        out_specs=o_spec,
        scratch_shapes=[
            pltpu.VMEM((N, C_in, L), jnp.bfloat16),
            pltpu.VMEM((C_in + 8, C_in), jnp.float32),
            pltpu.VMEM((C_out, C_in), jnp.bfloat16),
            pltpu.VMEM((C_out, 1), jnp.float32),
        ],
        compiler_params=pltpu.CompilerParams(
            dimension_semantics=("arbitrary", "arbitrary"),
            vmem_limit_bytes=_VMEM_LIMIT),
    )(x, w32, bn_gamma.reshape(C_out, 1), bn_beta.reshape(C_out, 1))
    return out


# speedup vs baseline: 1.2745x; 1.0084x over previous
import functools

import jax
import jax.numpy as jnp
from jax.experimental import pallas as pl
from jax.experimental.pallas import tpu as pltpu

_BN_EPS = 1e-5
_VMEM_LIMIT = 58 << 20


def _fused_kernel(x_ref, w_hbm, g_hbm, b_hbm, o_ref,
                  xs_ref, gacc_ref, w_scr, g_scr, b_scr, wsc_ref, shift_ref,
                  sems, *, nb, c, r):
    ph = pl.program_id(0)
    s = pl.program_id(1)

    @pl.when((ph == 0) & (s == 0))
    def _prefetch():
        pltpu.make_async_copy(w_hbm, w_scr, sems.at[0]).start()
        pltpu.make_async_copy(g_hbm, g_scr, sems.at[1]).start()
        pltpu.make_async_copy(b_hbm, b_scr, sems.at[2]).start()

    @pl.when(ph == 0)
    def _phase0():
        xs = []
        for b in range(nb):
            x16 = x_ref[b].astype(jnp.bfloat16)
            xs_ref[pl.ds(s * nb + b, 1)] = x16[None]
            xs.append(x16)
        xx = jnp.concatenate(xs, axis=1) if nb > 1 else xs[0]
        ones = jnp.ones((8, xx.shape[1]), jnp.bfloat16)
        xaug = jnp.concatenate([xx, ones], axis=0)
        g = jax.lax.dot_general(xaug, xx, (((1,), (1,)), ((), ())),
                                preferred_element_type=jnp.float32)

        @pl.when(s == 0)
        def _():
            gacc_ref[...] = g

        @pl.when(s != 0)
        def _():
            gacc_ref[...] += g

    @pl.when((ph == 1) & (s == 0))
    def _fold():
        pltpu.make_async_copy(w_hbm, w_scr, sems.at[0]).wait()
        pltpu.make_async_copy(g_hbm, g_scr, sems.at[1]).wait()
        pltpu.make_async_copy(b_hbm, b_scr, sems.at[2]).wait()
        w32 = w_scr[...]
        w16 = w32.astype(jnp.bfloat16)
        gram = gacc_ref[...]
        wga = jax.lax.dot_general(
            w16, gram.astype(jnp.bfloat16),
            (((1,), (1,)), ((), ())), preferred_element_type=jnp.float32)
        mean = wga[:, c:c + 1] / r
        ey2 = jnp.sum(wga[:, :c] * w32, axis=1, keepdims=True) / r
        var = jnp.maximum(ey2 - mean * mean, 0.0)
        scale = g_scr[...] * jax.lax.rsqrt(var + _BN_EPS)
        shift_ref[...] = b_scr[...] - mean * scale
        wsc_ref[...] = (w32 * scale).astype(jnp.bfloat16)

    @pl.when(ph == 1)
    def _phase1():
        wsc = wsc_ref[...]
        shift = shift_ref[...]
        for b in range(nb):
            x16 = xs_ref[s * nb + b]
            y = jnp.dot(wsc, x16, preferred_element_type=jnp.float32)
            o_ref[b] = jnp.maximum(y + shift, 0.0) + x16.astype(jnp.float32)


def kernel(x, conv_w, conv_b, bn_gamma, bn_beta):
    del conv_b
    N, C_in, L = x.shape
    C_out = conv_w.shape[0]
    w32 = conv_w[:, :, 0].astype(jnp.float32)

    nb = 2 if N % 2 == 0 else 1
    steps = N // nb

    x_spec = pl.BlockSpec(
        (nb, C_in, L), lambda ph, s: (jnp.where(ph == 0, s, steps - 1), 0, 0))
    o_spec = pl.BlockSpec(
        (nb, C_out, L), lambda ph, s: (jnp.where(ph == 0, 0, s), 0, 0))
    any_spec = pl.BlockSpec(memory_space=pl.ANY)

    out = pl.pallas_call(
        functools.partial(_fused_kernel, nb=nb, c=C_in, r=float(N * L)),
        out_shape=jax.ShapeDtypeStruct((N, C_out, L), x.dtype),
        grid=(2, steps),
        in_specs=[x_spec, any_spec, any_spec, any_spec],
        out_specs=o_spec,
        scratch_shapes=[
            pltpu.VMEM((N, C_in, L), jnp.bfloat16),
            pltpu.VMEM((C_in + 8, C_in), jnp.float32),
            pltpu.VMEM((C_out, C_in), jnp.float32),
            pltpu.VMEM((C_out, 1), jnp.float32),
            pltpu.VMEM((C_out, 1), jnp.float32),
            pltpu.VMEM((C_out, C_in), jnp.bfloat16),
            pltpu.VMEM((C_out, 1), jnp.float32),
            pltpu.SemaphoreType.DMA((3,)),
        ],
        compiler_params=pltpu.CompilerParams(
            dimension_semantics=("arbitrary", "arbitrary"),
            vmem_limit_bytes=_VMEM_LIMIT),
    )(x, w32, bn_gamma.reshape(C_out, 1), bn_beta.reshape(C_out, 1))
    return out


# speedup vs baseline: 1.3556x; 1.0636x over previous
import functools

import jax
import jax.numpy as jnp
from jax.experimental import pallas as pl
from jax.experimental.pallas import tpu as pltpu

_BN_EPS = 1e-5
_VMEM_LIMIT = 60000 << 10


def _fused_kernel(x_ref, w_hbm, gb_hbm, o_ref,
                  xs_ref, gacc_ref, w_scr, gb_scr,
                  sems, *, nb0, nb1, s0, c, r):
    s = pl.program_id(0)

    @pl.when(s == 0)
    def _prefetch():
        pltpu.make_async_copy(w_hbm, w_scr, sems.at[0]).start()
        pltpu.make_async_copy(gb_hbm, gb_scr, sems.at[1]).start()

    @pl.when(s < s0)
    def _phase0():
        xs = []
        for b in range(nb0):
            x16 = x_ref[b].astype(jnp.bfloat16)
            xs_ref[pl.ds(s * nb0 + b, 1)] = x16[None]
            xs.append(x16)
        xx = jnp.concatenate(xs, axis=1) if nb0 > 1 else xs[0]
        ones = jnp.ones((8, xx.shape[1]), jnp.bfloat16)
        xaug = jnp.concatenate([xx, ones], axis=0)
        g = jax.lax.dot_general(xaug, xx, (((1,), (1,)), ((), ())),
                                preferred_element_type=jnp.float32)

        @pl.when(s == 0)
        def _():
            gacc_ref[...] = g

        @pl.when(s != 0)
        def _():
            gacc_ref[...] += g

    @pl.when(s == s0)
    def _fold():
        pltpu.make_async_copy(w_hbm, w_scr, sems.at[0]).wait()
        pltpu.make_async_copy(gb_hbm, gb_scr, sems.at[1]).wait()
        w16 = w_scr[...]
        w32 = w16.astype(jnp.float32)
        gram = gacc_ref[...]
        wga = jax.lax.dot_general(
            w16, gram.astype(jnp.bfloat16),
            (((1,), (1,)), ((), ())), preferred_element_type=jnp.float32)
        mean = wga[:, c:c + 1] / r
        ey2 = jnp.sum(wga[:, :c] * w32, axis=1, keepdims=True) / r
        var = jnp.maximum(ey2 - mean * mean, 0.0)
        scale = gb_scr[:, 0:1] * jax.lax.rsqrt(var + _BN_EPS)
        shift = gb_scr[:, 1:2] - mean * scale
        gb_scr[:, 0:1] = shift
        w_scr[...] = (w32 * scale).astype(jnp.bfloat16)

    @pl.when(s >= s0)
    def _phase1():
        wsc = w_scr[...]
        shift = gb_scr[:, 0:1]
        for b in range(nb1):
            x16 = xs_ref[(s - s0) * nb1 + b]
            y = jnp.dot(wsc, x16, preferred_element_type=jnp.float32)
            o_ref[b] = jnp.maximum(y + shift, 0.0) + x16.astype(jnp.float32)


def kernel(x, conv_w, conv_b, bn_gamma, bn_beta):
    del conv_b
    N, C_in, L = x.shape
    C_out = conv_w.shape[0]
    w16 = conv_w[:, :, 0].astype(jnp.bfloat16)

    nb0 = 2 if N % 2 == 0 else 1
    nb1 = 4 if N % 4 == 0 else nb0
    s0 = N // nb0
    s1 = N // nb1

    x_spec = pl.BlockSpec(
        (nb0, C_in, L), lambda s: (jnp.where(s < s0, s, s0 - 1), 0, 0))
    o_spec = pl.BlockSpec(
        (nb1, C_out, L), lambda s: (jnp.where(s < s0, 0, s - s0), 0, 0))
    any_spec = pl.BlockSpec(memory_space=pl.ANY)

    out = pl.pallas_call(
        functools.partial(_fused_kernel, nb0=nb0, nb1=nb1, s0=s0, c=C_in,
                          r=float(N * L)),
        out_shape=jax.ShapeDtypeStruct((N, C_out, L), x.dtype),
        grid=(s0 + s1,),
        in_specs=[x_spec, any_spec, any_spec],
        out_specs=o_spec,
        scratch_shapes=[
            pltpu.VMEM((N, C_in, L), jnp.bfloat16),
            pltpu.VMEM((C_in + 8, C_in), jnp.float32),
            pltpu.VMEM((C_out, C_in), jnp.bfloat16),
            pltpu.VMEM((C_out, 2), jnp.float32),
            pltpu.SemaphoreType.DMA((2,)),
        ],
        compiler_params=pltpu.CompilerParams(
            dimension_semantics=("arbitrary",),
            vmem_limit_bytes=_VMEM_LIMIT),
    )(x, w16, jnp.stack([bn_gamma, bn_beta], axis=1))
    return out
